# Initial kernel scaffold; baseline (speedup 1.0000x reference)
import jax
import jax.numpy as jnp
from jax.experimental import pallas as pl


def kernel(x, edge_index, W1, b1, W2, b2):
    raise NotImplementedError("write your pallas kernel here")


# speedup vs baseline: 12.8534x; 12.8534x over previous
import functools

import jax
import jax.numpy as jnp
from jax import lax
from jax.experimental import pallas as pl
from jax.experimental.pallas import tpu as pltpu
from jax.experimental.pallas import tpu_sc as plsc

_N = 10000
_D = 128
_E = 320000
_NC = 2
_NS = 16
_NW = _NC * _NS
_EPW = _E // _NW
_C = 80
_NCHUNK = _EPW // _C
_NP = 10240
_RPT = _NP // _NS
_CH = 128

_mesh = plsc.VectorSubcoreMesh(core_axis_name="c", subcore_axis_name="s")


def _deg_body(dst_hbm, zd_hbm, out_hbm, dst_v, ones_v, dbuf, accd):
    c = lax.axis_index("c")
    s = lax.axis_index("s")
    wid = s * _NC + c
    for k in range(_C // 16):
        ones_v[pl.ds(k * 16, 16)] = jnp.ones((16,), jnp.float32)
    pltpu.sync_copy(zd_hbm, dbuf)
    pltpu.sync_copy(dbuf, accd.at[pl.ds(s * _RPT, _RPT)])
    plsc.subcore_barrier()
    base = wid * _EPW

    def body(i, carry):
        off = pl.multiple_of(base + i * _C, 8)
        pltpu.sync_copy(dst_hbm.at[pl.ds(off, _C)], dst_v)
        pltpu.sync_copy(ones_v, accd.at[dst_v], add=True)
        return carry

    lax.fori_loop(0, _NCHUNK, body, 0)
    plsc.subcore_barrier()
    pltpu.sync_copy(accd.at[pl.ds(s * _RPT, _RPT)], dbuf)
    pltpu.sync_copy(dbuf, out_hbm.at[pl.ds(c * _NP + s * _RPT, _RPT)])


_deg_kernel = pl.kernel(
    _deg_body,
    out_type=jax.ShapeDtypeStruct((_NC * _NP,), jnp.float32),
    mesh=_mesh,
    scratch_types=[
        pltpu.VMEM((_C,), jnp.int32),
        pltpu.VMEM((_C,), jnp.float32),
        pltpu.VMEM((_RPT,), jnp.float32),
        pltpu.VMEM_SHARED((_NP,), jnp.float32),
    ],
)


def _agg_body(yp_hbm, src_hbm, dst_hbm, z2_hbm, out_hbm,
              src_v, dst_v, rows_v, zbuf, acc, sem):
    c = lax.axis_index("c")
    s = lax.axis_index("s")
    wid = s * _NC + c
    pltpu.sync_copy(z2_hbm, zbuf)
    for k in range(_RPT // _CH):
        pltpu.sync_copy(zbuf, acc.at[pl.ds(s * _RPT + k * _CH, _CH)])
    plsc.subcore_barrier()
    base = wid * _EPW

    def body(i, carry):
        off = pl.multiple_of(base + i * _C, 8)
        pltpu.sync_copy(src_hbm.at[pl.ds(off, _C)], src_v)
        pltpu.sync_copy(dst_hbm.at[pl.ds(off, _C)], dst_v)
        pltpu.async_copy(yp_hbm.at[src_v], rows_v, sem).wait()
        pltpu.sync_copy(rows_v, acc.at[dst_v], add=True)
        return carry

    lax.fori_loop(0, _NCHUNK, body, 0)
    plsc.subcore_barrier()
    for k in range(_RPT // _CH):
        pltpu.sync_copy(acc.at[pl.ds(s * _RPT + k * _CH, _CH)], zbuf)
        pltpu.sync_copy(zbuf, out_hbm.at[c, pl.ds(s * _RPT + k * _CH, _CH)])


_agg_kernel = pl.kernel(
    _agg_body,
    out_type=jax.ShapeDtypeStruct((_NC, _NP, _D), jnp.float32),
    mesh=_mesh,
    scratch_types=[
        pltpu.VMEM((_C,), jnp.int32),
        pltpu.VMEM((_C,), jnp.int32),
        pltpu.VMEM((_C, _D), jnp.float32),
        pltpu.VMEM((_CH, _D), jnp.float32),
        pltpu.VMEM_SHARED((_NP, _D), jnp.float32),
        pltpu.SemaphoreType.DMA,
    ],
)


_R = 1000


def _pre_body(x_ref, w_ref, degp_ref, xw_ref, yp_ref):
    dinv = lax.rsqrt(degp_ref[0] + degp_ref[1] + 1.0)
    xw = jnp.dot(x_ref[...], w_ref[...], preferred_element_type=jnp.float32)
    xw_ref[...] = xw
    yp_ref[...] = xw * dinv


def _mid_body(p_ref, xw1_ref, degp_ref, b_ref, w_ref, xw2_ref, yp_ref):
    dinv = lax.rsqrt(degp_ref[0] + degp_ref[1] + 1.0)
    z = ((p_ref[0] + p_ref[1]) * dinv
         + xw1_ref[...] * (dinv * dinv) + b_ref[...])
    h = jnp.maximum(z, 0.0)
    xw2 = jnp.dot(h, w_ref[...], preferred_element_type=jnp.float32)
    xw2_ref[...] = xw2
    yp_ref[...] = xw2 * dinv


def _fin_body(p_ref, xw2_ref, degp_ref, b_ref, out_ref):
    dinv = lax.rsqrt(degp_ref[0] + degp_ref[1] + 1.0)
    out_ref[...] = ((p_ref[0] + p_ref[1]) * dinv
                    + xw2_ref[...] * (dinv * dinv) + b_ref[...])


_spec_rows = pl.BlockSpec((_R, _D), lambda i: (i, 0))
_spec_w = pl.BlockSpec((_D, _D), lambda i: (0, 0))
_spec_degp = pl.BlockSpec((2, _R, 1), lambda i: (0, i, 0))
_spec_part = pl.BlockSpec((2, _R, _D), lambda i: (0, i, 0))
_spec_b = pl.BlockSpec((1, _D), lambda i: (0, 0))
_sds_rows = jax.ShapeDtypeStruct((_N, _D), jnp.float32)

_pre_call = pl.pallas_call(
    _pre_body,
    grid=(_N // _R,),
    in_specs=[_spec_rows, _spec_w, _spec_degp],
    out_specs=[_spec_rows, _spec_rows],
    out_shape=[_sds_rows, _sds_rows],
)

_mid_call = pl.pallas_call(
    _mid_body,
    grid=(_N // _R,),
    in_specs=[_spec_part, _spec_rows, _spec_degp, _spec_b, _spec_w],
    out_specs=[_spec_rows, _spec_rows],
    out_shape=[_sds_rows, _sds_rows],
)

_fin_call = pl.pallas_call(
    _fin_body,
    grid=(_N // _R,),
    in_specs=[_spec_part, _spec_rows, _spec_degp, _spec_b],
    out_specs=_spec_rows,
    out_shape=_sds_rows,
)


def kernel(x, edge_index, W1, b1, W2, b2):
    ei = edge_index.astype(jnp.int32)
    src = ei[0]
    dst = ei[1]
    zeros2d = jnp.zeros((_CH, _D), jnp.float32)
    zerosd = jnp.zeros((_RPT,), jnp.float32)

    degp = _deg_kernel(dst, zerosd)
    degp3 = degp.reshape(_NC, _NP, 1)[:, :_N]

    xw1, y1p = _pre_call(x, W1, degp3)
    p1 = _agg_kernel(y1p, src, dst, zeros2d)[:, :_N]
    xw2, y2p = _mid_call(p1, xw1, degp3, b1.reshape(1, _D), W2)
    p2 = _agg_kernel(y2p, src, dst, zeros2d)[:, :_N]
    out = _fin_call(p2, xw2, degp3, b2.reshape(1, _D))
    return out


# speedup vs baseline: 24.6373x; 1.9168x over previous
import functools

import jax
import jax.numpy as jnp
from jax import lax
from jax.experimental import pallas as pl
from jax.experimental.pallas import tpu as pltpu
from jax.experimental.pallas import tpu_sc as plsc

_N = 10000
_D = 128
_E = 320000
_NC = 2
_NS = 16
_NW = _NC * _NS
_EPW = _E // _NW
_C = 80
_NCHUNK = _EPW // _C
_NP = 10240
_RPT = _NP // _NS
_CH = 128

_mesh = plsc.VectorSubcoreMesh(core_axis_name="c", subcore_axis_name="s")


def _deg_body(dst3_hbm, zd_hbm, out_hbm, dst_big, ones_v, dbuf, accd):
    c = lax.axis_index("c")
    s = lax.axis_index("s")
    wid = s * _NC + c
    for k in range(_C // 16):
        ones_v[pl.ds(k * 16, 16)] = jnp.ones((16,), jnp.float32)
    pltpu.sync_copy(dst3_hbm.at[wid], dst_big)
    pltpu.sync_copy(zd_hbm, dbuf)
    pltpu.sync_copy(dbuf, accd.at[pl.ds(s * _RPT, _RPT)])
    plsc.subcore_barrier()

    def body(i, carry):
        pltpu.sync_copy(ones_v, accd.at[dst_big.at[i]], add=True)
        return carry

    lax.fori_loop(0, _NCHUNK, body, 0)
    plsc.subcore_barrier()
    pltpu.sync_copy(accd.at[pl.ds(s * _RPT, _RPT)], dbuf)
    pltpu.sync_copy(dbuf, out_hbm.at[pl.ds(c * _NP + s * _RPT, _RPT)])


_deg_kernel = pl.kernel(
    _deg_body,
    out_type=jax.ShapeDtypeStruct((_NC * _NP,), jnp.float32),
    mesh=_mesh,
    scratch_types=[
        pltpu.VMEM((_NCHUNK, _C), jnp.int32),
        pltpu.VMEM((_C,), jnp.float32),
        pltpu.VMEM((_RPT,), jnp.float32),
        pltpu.VMEM_SHARED((_NP,), jnp.float32),
    ],
)


def _agg_body(yp_hbm, src_hbm, dst_hbm, z2_hbm, out_hbm,
              src_v0, dst_v0, src_v1, dst_v1, rows_v0, rows_v1,
              acc, semi0, semi1, semg0, semg1):
    c = lax.axis_index("c")
    s = lax.axis_index("s")
    wid = s * _NC + c
    pltpu.sync_copy(z2_hbm, rows_v0)
    for k in range(_RPT // _C):
        pltpu.sync_copy(rows_v0, acc.at[pl.ds(s * _RPT + k * _C, _C)])
    plsc.subcore_barrier()
    base = wid * _EPW
    ibufs = ((src_v0, dst_v0, semi0), (src_v1, dst_v1, semi1))

    def fetch_idx(i, b):
        src_v, dst_v, semi = ibufs[b]
        i = jnp.minimum(i, _NCHUNK - 1)
        off = pl.multiple_of(base + i * _C, 8)
        pltpu.async_copy(src_hbm.at[pl.ds(off, _C)], src_v, semi)
        pltpu.async_copy(dst_hbm.at[pl.ds(off, _C)], dst_v, semi)

    def wait_idx(b):
        src_v, dst_v, semi = ibufs[b]
        pltpu.make_async_copy(src_hbm.at[pl.ds(0, _C)], src_v, semi).wait()
        pltpu.make_async_copy(dst_hbm.at[pl.ds(0, _C)], dst_v, semi).wait()

    fetch_idx(0, 0)
    wait_idx(0)
    pltpu.async_copy(yp_hbm.at[src_v0], rows_v0, semg0)
    fetch_idx(1, 1)

    def body(k, carry):
        i0 = 2 * k
        pltpu.make_async_copy(yp_hbm.at[src_v0], rows_v0, semg0).wait()
        wait_idx(1)
        pltpu.async_copy(yp_hbm.at[src_v1], rows_v1, semg1)
        pltpu.sync_copy(rows_v0, acc.at[dst_v0], add=True)
        fetch_idx(i0 + 2, 0)
        pltpu.make_async_copy(yp_hbm.at[src_v1], rows_v1, semg1).wait()
        wait_idx(0)
        pltpu.async_copy(yp_hbm.at[src_v0], rows_v0, semg0)
        pltpu.sync_copy(rows_v1, acc.at[dst_v1], add=True)
        fetch_idx(i0 + 3, 1)
        return carry

    lax.fori_loop(0, (_NCHUNK - 1) // 2, body, 0)
    pltpu.make_async_copy(yp_hbm.at[src_v0], rows_v0, semg0).wait()
    pltpu.sync_copy(rows_v0, acc.at[dst_v0], add=True)
    wait_idx(1)
    plsc.subcore_barrier()
    for k in range(_RPT // _C):
        pltpu.sync_copy(acc.at[pl.ds(s * _RPT + k * _C, _C)], rows_v0)
        pltpu.sync_copy(rows_v0, out_hbm.at[c, pl.ds(s * _RPT + k * _C, _C)])


_agg_kernel = pl.kernel(
    _agg_body,
    out_type=jax.ShapeDtypeStruct((_NC, _NP, _D), jnp.float32),
    mesh=_mesh,
    scratch_types=[
        pltpu.VMEM((_C,), jnp.int32),
        pltpu.VMEM((_C,), jnp.int32),
        pltpu.VMEM((_C,), jnp.int32),
        pltpu.VMEM((_C,), jnp.int32),
        pltpu.VMEM((_C, _D), jnp.float32),
        pltpu.VMEM((_C, _D), jnp.float32),
        pltpu.VMEM_SHARED((_NP, _D), jnp.float32),
        pltpu.SemaphoreType.DMA,
        pltpu.SemaphoreType.DMA,
        pltpu.SemaphoreType.DMA,
        pltpu.SemaphoreType.DMA,
    ],
)


_R = 1000


def _pre_body(x_ref, w_ref, degp_ref, xw_ref, yp_ref):
    dinv = lax.rsqrt(degp_ref[0] + degp_ref[1] + 1.0)
    xw = jnp.dot(x_ref[...], w_ref[...], preferred_element_type=jnp.float32)
    xw_ref[...] = xw
    yp_ref[...] = xw * dinv


def _mid_body(p_ref, xw1_ref, degp_ref, b_ref, w_ref, xw2_ref, yp_ref):
    dinv = lax.rsqrt(degp_ref[0] + degp_ref[1] + 1.0)
    z = ((p_ref[0] + p_ref[1]) * dinv
         + xw1_ref[...] * (dinv * dinv) + b_ref[...])
    h = jnp.maximum(z, 0.0)
    xw2 = jnp.dot(h, w_ref[...], preferred_element_type=jnp.float32)
    xw2_ref[...] = xw2
    yp_ref[...] = xw2 * dinv


def _fin_body(p_ref, xw2_ref, degp_ref, b_ref, out_ref):
    dinv = lax.rsqrt(degp_ref[0] + degp_ref[1] + 1.0)
    out_ref[...] = ((p_ref[0] + p_ref[1]) * dinv
                    + xw2_ref[...] * (dinv * dinv) + b_ref[...])


_spec_rows = pl.BlockSpec((_R, _D), lambda i: (i, 0))
_spec_w = pl.BlockSpec((_D, _D), lambda i: (0, 0))
_spec_degp = pl.BlockSpec((2, _R, 1), lambda i: (0, i, 0))
_spec_part = pl.BlockSpec((2, _R, _D), lambda i: (0, i, 0))
_spec_b = pl.BlockSpec((1, _D), lambda i: (0, 0))
_sds_rows = jax.ShapeDtypeStruct((_N, _D), jnp.float32)

_pre_call = pl.pallas_call(
    _pre_body,
    grid=(_N // _R,),
    in_specs=[_spec_rows, _spec_w, _spec_degp],
    out_specs=[_spec_rows, _spec_rows],
    out_shape=[_sds_rows, _sds_rows],
)

_mid_call = pl.pallas_call(
    _mid_body,
    grid=(_N // _R,),
    in_specs=[_spec_part, _spec_rows, _spec_degp, _spec_b, _spec_w],
    out_specs=[_spec_rows, _spec_rows],
    out_shape=[_sds_rows, _sds_rows],
)

_fin_call = pl.pallas_call(
    _fin_body,
    grid=(_N // _R,),
    in_specs=[_spec_part, _spec_rows, _spec_degp, _spec_b],
    out_specs=_spec_rows,
    out_shape=_sds_rows,
)


def kernel(x, edge_index, W1, b1, W2, b2):
    ei = edge_index.astype(jnp.int32)
    src = ei[0]
    dst = ei[1]
    dst3 = dst.reshape(_NW, _NCHUNK, _C)
    zeros2d = jnp.zeros((_C, _D), jnp.float32)
    zerosd = jnp.zeros((_RPT,), jnp.float32)

    degp = _deg_kernel(dst3, zerosd)
    degp3 = degp.reshape(_NC, _NP, 1)

    xw1, y1p = _pre_call(x, W1, degp3)
    p1 = _agg_kernel(y1p, src, dst, zeros2d)
    xw2, y2p = _mid_call(p1, xw1, degp3, b1.reshape(1, _D), W2)
    p2 = _agg_kernel(y2p, src, dst, zeros2d)
    out = _fin_call(p2, xw2, degp3, b2.reshape(1, _D))
    return out


# speedup vs baseline: 29.3759x; 1.1923x over previous
import functools

import jax
import jax.numpy as jnp
from jax import lax
from jax.experimental import pallas as pl
from jax.experimental.pallas import tpu as pltpu
from jax.experimental.pallas import tpu_sc as plsc

_N = 10000
_D = 128
_E = 320000
_NC = 2
_NS = 16
_NW = _NC * _NS
_EPW = _E // _NW
_C = 128
_NCHUNK = _EPW // _C
_TAIL = _EPW - _NCHUNK * _C
_NP = 10240
_RPT = _NP // _NS
_DEG_W = 4

_mesh = plsc.VectorSubcoreMesh(core_axis_name="c", subcore_axis_name="s")


def _deg_body(pk_hbm, pkt_hbm, zd_hbm, out_hbm,
              idx_big, pkt_v, ones_v, ones_t, dbuf, accd, semd):
    c = lax.axis_index("c")
    s = lax.axis_index("s")
    wid = s * _NC + c
    for k in range(_C // 16):
        ones_v[pl.ds(k * 16, 16)] = jnp.ones((16,), jnp.float32)
    ones_t[...] = jnp.ones((_TAIL,), jnp.float32)
    pltpu.sync_copy(pk_hbm.at[wid], idx_big)
    pltpu.sync_copy(pkt_hbm.at[wid], pkt_v)
    pltpu.sync_copy(zd_hbm, dbuf)
    pltpu.sync_copy(dbuf, accd.at[pl.ds(s * _RPT, _RPT)])
    plsc.subcore_barrier()

    def fire(j):
        pltpu.async_copy(ones_v, accd.at[idx_big.at[j, 1]], semd, add=True)

    def wait_one():
        pltpu.make_async_copy(ones_v, accd.at[idx_big.at[0, 1]], semd).wait()

    for j in range(_DEG_W):
        fire(j)

    def body(j, carry):
        wait_one()
        fire(j + _DEG_W)
        return carry

    lax.fori_loop(0, _NCHUNK - _DEG_W, body, 0)
    for _ in range(_DEG_W):
        wait_one()
    pltpu.sync_copy(ones_t, accd.at[pkt_v.at[1]], add=True)
    plsc.subcore_barrier()
    pltpu.sync_copy(accd.at[pl.ds(s * _RPT, _RPT)], dbuf)
    pltpu.sync_copy(dbuf, out_hbm.at[pl.ds(c * _NP + s * _RPT, _RPT)])


_deg_kernel = pl.kernel(
    _deg_body,
    out_type=jax.ShapeDtypeStruct((_NC * _NP,), jnp.float32),
    mesh=_mesh,
    scratch_types=[
        pltpu.VMEM((_NCHUNK, 2, _C), jnp.int32),
        pltpu.VMEM((2, _TAIL), jnp.int32),
        pltpu.VMEM((_C,), jnp.float32),
        pltpu.VMEM((_TAIL,), jnp.float32),
        pltpu.VMEM((_RPT,), jnp.float32),
        pltpu.VMEM_SHARED((_NP,), jnp.float32),
        pltpu.SemaphoreType.DMA,
    ],
)


def _agg_body(yp_hbm, pk_hbm, pkt_hbm, z2_hbm, out_hbm,
              pk_v0, pk_v1, pkt_v, rows_v0, rows_v1,
              acc, semi0, semi1, semg0, semg1):
    c = lax.axis_index("c")
    s = lax.axis_index("s")
    wid = s * _NC + c
    pltpu.sync_copy(z2_hbm, rows_v0)
    for k in range(_RPT // _C):
        pltpu.async_copy(rows_v0, acc.at[pl.ds(s * _RPT + k * _C, _C)], semg0)
    for k in range(_RPT // _C):
        pltpu.make_async_copy(rows_v0, acc.at[pl.ds(0, _C)], semg0).wait()
    plsc.subcore_barrier()
    ibufs = ((pk_v0, semi0), (pk_v1, semi1))

    def fetch_idx(i, b):
        pk_v, semi = ibufs[b]
        i = jnp.minimum(i, _NCHUNK - 1)
        pltpu.async_copy(pk_hbm.at[wid, i], pk_v, semi)

    def wait_idx(b):
        pk_v, semi = ibufs[b]
        pltpu.make_async_copy(pk_hbm.at[0, 0], pk_v, semi).wait()

    fetch_idx(0, 0)
    wait_idx(0)
    pltpu.async_copy(yp_hbm.at[pk_v0.at[0]], rows_v0, semg0)
    fetch_idx(1, 1)

    def body(k, carry):
        i0 = 2 * k
        pltpu.make_async_copy(yp_hbm.at[pk_v0.at[0]], rows_v0, semg0).wait()
        wait_idx(1)
        pltpu.async_copy(yp_hbm.at[pk_v1.at[0]], rows_v1, semg1)
        pltpu.sync_copy(rows_v0, acc.at[pk_v0.at[1]], add=True)
        fetch_idx(i0 + 2, 0)
        pltpu.make_async_copy(yp_hbm.at[pk_v1.at[0]], rows_v1, semg1).wait()
        wait_idx(0)
        pltpu.async_copy(yp_hbm.at[pk_v0.at[0]], rows_v0, semg0)
        pltpu.sync_copy(rows_v1, acc.at[pk_v1.at[1]], add=True)
        fetch_idx(i0 + 3, 1)
        return carry

    lax.fori_loop(0, _NCHUNK // 2, body, 0)
    pltpu.make_async_copy(yp_hbm.at[pk_v0.at[0]], rows_v0, semg0).wait()
    wait_idx(1)
    pltpu.sync_copy(pkt_hbm.at[wid], pkt_v)
    pltpu.async_copy(yp_hbm.at[pkt_v.at[0]], rows_v0.at[pl.ds(0, _TAIL)],
                     semg0).wait()
    pltpu.sync_copy(rows_v0.at[pl.ds(0, _TAIL)], acc.at[pkt_v.at[1]], add=True)
    plsc.subcore_barrier()
    wbufs = (rows_v0, rows_v1)
    pltpu.async_copy(acc.at[pl.ds(s * _RPT, _C)], rows_v0, semg0)
    for k in range(_RPT // _C):
        buf = wbufs[k % 2]
        sem = (semg0, semg1)[k % 2]
        pltpu.make_async_copy(acc.at[pl.ds(0, _C)], buf, sem).wait()
        if k + 1 < _RPT // _C:
            pltpu.async_copy(acc.at[pl.ds(s * _RPT + (k + 1) * _C, _C)],
                             wbufs[(k + 1) % 2], (semg0, semg1)[(k + 1) % 2])
        pltpu.sync_copy(buf, out_hbm.at[c, pl.ds(s * _RPT + k * _C, _C)])


_agg_kernel = pl.kernel(
    _agg_body,
    out_type=jax.ShapeDtypeStruct((_NC, _NP, _D), jnp.float32),
    mesh=_mesh,
    scratch_types=[
        pltpu.VMEM((2, _C), jnp.int32),
        pltpu.VMEM((2, _C), jnp.int32),
        pltpu.VMEM((2, _TAIL), jnp.int32),
        pltpu.VMEM((_C, _D), jnp.float32),
        pltpu.VMEM((_C, _D), jnp.float32),
        pltpu.VMEM_SHARED((_NP, _D), jnp.float32),
        pltpu.SemaphoreType.DMA,
        pltpu.SemaphoreType.DMA,
        pltpu.SemaphoreType.DMA,
        pltpu.SemaphoreType.DMA,
    ],
)


_R = 1000


def _pre_body(x_ref, w_ref, degp_ref, xw_ref, yp_ref):
    dinv = lax.rsqrt(degp_ref[0] + degp_ref[1] + 1.0)
    xw = jnp.dot(x_ref[...], w_ref[...], preferred_element_type=jnp.float32)
    xw_ref[...] = xw
    yp_ref[...] = xw * dinv


def _mid_body(p_ref, xw1_ref, degp_ref, b_ref, w_ref, xw2_ref, yp_ref):
    dinv = lax.rsqrt(degp_ref[0] + degp_ref[1] + 1.0)
    z = ((p_ref[0] + p_ref[1]) * dinv
         + xw1_ref[...] * (dinv * dinv) + b_ref[...])
    h = jnp.maximum(z, 0.0)
    xw2 = jnp.dot(h, w_ref[...], preferred_element_type=jnp.float32)
    xw2_ref[...] = xw2
    yp_ref[...] = xw2 * dinv


def _fin_body(p_ref, xw2_ref, degp_ref, b_ref, out_ref):
    dinv = lax.rsqrt(degp_ref[0] + degp_ref[1] + 1.0)
    out_ref[...] = ((p_ref[0] + p_ref[1]) * dinv
                    + xw2_ref[...] * (dinv * dinv) + b_ref[...])


_spec_rows = pl.BlockSpec((_R, _D), lambda i: (i, 0))
_spec_w = pl.BlockSpec((_D, _D), lambda i: (0, 0))
_spec_degp = pl.BlockSpec((2, _R, 1), lambda i: (0, i, 0))
_spec_part = pl.BlockSpec((2, _R, _D), lambda i: (0, i, 0))
_spec_b = pl.BlockSpec((1, _D), lambda i: (0, 0))
_sds_rows = jax.ShapeDtypeStruct((_N, _D), jnp.float32)

_pre_call = pl.pallas_call(
    _pre_body,
    grid=(_N // _R,),
    in_specs=[_spec_rows, _spec_w, _spec_degp],
    out_specs=[_spec_rows, _spec_rows],
    out_shape=[_sds_rows, _sds_rows],
)

_mid_call = pl.pallas_call(
    _mid_body,
    grid=(_N // _R,),
    in_specs=[_spec_part, _spec_rows, _spec_degp, _spec_b, _spec_w],
    out_specs=[_spec_rows, _spec_rows],
    out_shape=[_sds_rows, _sds_rows],
)

_fin_call = pl.pallas_call(
    _fin_body,
    grid=(_N // _R,),
    in_specs=[_spec_part, _spec_rows, _spec_degp, _spec_b],
    out_specs=_spec_rows,
    out_shape=_sds_rows,
)


def kernel(x, edge_index, W1, b1, W2, b2):
    ei = edge_index.astype(jnp.int32)
    eiw = ei.reshape(2, _NW, _EPW)
    pk = jnp.stack([eiw[0, :, :_NCHUNK * _C].reshape(_NW, _NCHUNK, _C),
                    eiw[1, :, :_NCHUNK * _C].reshape(_NW, _NCHUNK, _C)],
                   axis=2)
    pkt = jnp.stack([eiw[0, :, _NCHUNK * _C:],
                     eiw[1, :, _NCHUNK * _C:]], axis=1)
    zeros2d = jnp.zeros((_C, _D), jnp.float32)
    zerosd = jnp.zeros((_RPT,), jnp.float32)

    degp = _deg_kernel(pk, pkt, zerosd)
    degp3 = degp.reshape(_NC, _NP, 1)

    xw1, y1p = _pre_call(x, W1, degp3)
    p1 = _agg_kernel(y1p, pk, pkt, zeros2d)
    xw2, y2p = _mid_call(p1, xw1, degp3, b1.reshape(1, _D), W2)
    p2 = _agg_kernel(y2p, pk, pkt, zeros2d)
    out = _fin_call(p2, xw2, degp3, b2.reshape(1, _D))
    return out
